# baseline (device time: 39991 ns/iter reference)
import jax
import jax.numpy as jnp
from jax import lax
from jax.experimental import pallas as pl
from jax.experimental.pallas import tpu as pltpu

N_DEV = 8
GELU_C = 0.7978845608028654


def _gelu(y):
    return 0.5 * y * (1.0 + jnp.tanh(GELU_C * (y + 0.044715 * y * y * y)))


def kernel(x, w_mat):
    m, k_per = x.shape
    _, n = w_mat.shape
    m_per = m // N_DEV

    def body(x_ref, w_ref, out_ref, p_ref, comm_ref, send_sems, recv_sems):
        my = lax.axis_index("i")
        left = lax.rem(my + N_DEV - 1, N_DEV)
        right = lax.rem(my + 1, N_DEV)

        barrier_sem = pltpu.get_barrier_semaphore()
        for nbr in (left, right):
            pl.semaphore_signal(
                barrier_sem, inc=1,
                device_id=(nbr,), device_id_type=pl.DeviceIdType.MESH,
            )
        pl.semaphore_wait(barrier_sem, 2)

        p_ref[:, :] = jnp.dot(
            x_ref[:, :], w_ref[:, :], preferred_element_type=jnp.float32
        )

        def p_chunk(c):
            return p_ref[pl.ds(c * m_per, m_per), :]

        c0 = lax.rem(my + N_DEV - 1, N_DEV)
        comm_ref[0, :, :] = p_chunk(c0).astype(jnp.bfloat16)

        for s in range(N_DEV - 1):
            rdma = pltpu.make_async_remote_copy(
                src_ref=comm_ref.at[s],
                dst_ref=comm_ref.at[s + 1],
                send_sem=send_sems.at[s],
                recv_sem=recv_sems.at[s],
                device_id=(right,),
                device_id_type=pl.DeviceIdType.MESH,
            )
            rdma.start()
            rdma.wait()

            c = lax.rem(my + 2 * N_DEV - s - 2, N_DEV)
            acc = comm_ref[s + 1, :, :].astype(jnp.float32) + p_chunk(c)
            if s < N_DEV - 2:
                comm_ref[s + 1, :, :] = acc.astype(jnp.bfloat16)
            else:
                out_ref[:, :] = _gelu(acc)

    return pl.pallas_call(
        body,
        out_shape=jax.ShapeDtypeStruct((m_per, n), jnp.float32),
        in_specs=[
            pl.BlockSpec(memory_space=pltpu.VMEM),
            pl.BlockSpec(memory_space=pltpu.VMEM),
        ],
        out_specs=pl.BlockSpec(memory_space=pltpu.VMEM),
        scratch_shapes=[
            pltpu.VMEM((m, n), jnp.float32),
            pltpu.VMEM((N_DEV, m_per, n), jnp.bfloat16),
            pltpu.SemaphoreType.DMA((N_DEV - 1,)),
            pltpu.SemaphoreType.DMA((N_DEV - 1,)),
        ],
        compiler_params=pltpu.CompilerParams(collective_id=0),
    )(x, w_mat)


# device time: 7248 ns/iter; 5.5175x vs baseline; 5.5175x over previous
import jax
import jax.numpy as jnp
from jax import lax
from jax.experimental import pallas as pl
from jax.experimental.pallas import tpu as pltpu

N_DEV = 8
M_PER = 128
GELU_C = 0.7978845608028654

PMASK = {"x": 1, "y": 3, "z": 4}

BIT = {
    "x": lambda v: (v & 1) ^ ((v >> 1) & 1),
    "y": lambda v: (v >> 1) & 1,
    "z": lambda v: (v >> 2) & 1,
}


def _chunk_of(bits):
    return bits["z"] * 4 + bits["y"] * 2 + (bits["x"] ^ bits["y"])


def _gelu(y):
    return 0.5 * y * (1.0 + jnp.tanh(GELU_C * (y + 0.044715 * y * y * y)))


STREAMS = [
    (0, 1024, ("z", "y", "x")),
]

SLOT_BASE = [0, 4, 6]


def kernel(x, w_mat):
    m, k_per = x.shape
    _, n = w_mat.shape

    n_streams = len(STREAMS)

    def body(x_ref, w_ref, out_ref, p_ref, *rest):
        send_bufs = rest[0:2 * n_streams:2]
        recv_bufs = rest[1:2 * n_streams:2]
        send_sems, recv_sems = rest[2 * n_streams], rest[2 * n_streams + 1]

        my = lax.axis_index("i")
        mybit = {a: BIT[a](my) for a in "xyz"}

        barrier_sem = pltpu.get_barrier_semaphore()
        for mask in (1, 3, 4):
            pl.semaphore_signal(
                barrier_sem, inc=1,
                device_id=(my ^ mask,), device_id_type=pl.DeviceIdType.MESH,
            )
        pl.semaphore_wait(barrier_sem, 3)

        p_ref[:, :] = jnp.dot(
            x_ref[:, :], w_ref[:, :], preferred_element_type=jnp.float32
        )

        for r in range(3):
            started = []
            for si, (c0, w, order) in enumerate(STREAMS):
                a = order[r]
                free = order[r + 1:]
                partner = my ^ PMASK[a]
                sb, rb = send_bufs[si], recv_bufs[si]
                for k in range(2 ** len(free)):
                    bits = dict(mybit)
                    for j, f in enumerate(free):
                        bits[f] = (k >> j) & 1
                    bits_send = dict(bits)
                    bits_send[a] = mybit[a] ^ 1
                    c_send = _chunk_of(bits_send)
                    c_recv = _chunk_of(bits)
                    slot = SLOT_BASE[r] + k
                    sb[slot, :, :] = p_ref[
                        pl.ds(c_send * M_PER, M_PER), c0:c0 + w
                    ].astype(jnp.bfloat16)
                    rdma = pltpu.make_async_remote_copy(
                        src_ref=sb.at[slot],
                        dst_ref=rb.at[slot],
                        send_sem=send_sems.at[si, slot],
                        recv_sem=recv_sems.at[si, slot],
                        device_id=(partner,),
                        device_id_type=pl.DeviceIdType.MESH,
                    )
                    rdma.start()
                    started.append((rdma, si, slot, c_recv, c0, w))
            for rdma, *_ in started:
                rdma.wait()
            for _, si, slot, c_recv, c0, w in started:
                rows = pl.ds(c_recv * M_PER, M_PER)
                p_ref[rows, c0:c0 + w] = (
                    p_ref[rows, c0:c0 + w]
                    + recv_bufs[si][slot, :, :].astype(jnp.float32)
                )

        out_ref[:, :] = _gelu(p_ref[pl.ds(my * M_PER, M_PER), :])

    scratch = [pltpu.VMEM((m, n), jnp.float32)]
    for c0, w, order in STREAMS:
        scratch.append(pltpu.VMEM((7, M_PER, w), jnp.bfloat16))
        scratch.append(pltpu.VMEM((7, M_PER, w), jnp.bfloat16))
    scratch.append(pltpu.SemaphoreType.DMA((n_streams, 7)))
    scratch.append(pltpu.SemaphoreType.DMA((n_streams, 7)))

    return pl.pallas_call(
        body,
        out_shape=jax.ShapeDtypeStruct((M_PER, n), jnp.float32),
        in_specs=[
            pl.BlockSpec(memory_space=pltpu.VMEM),
            pl.BlockSpec(memory_space=pltpu.VMEM),
        ],
        out_specs=pl.BlockSpec(memory_space=pltpu.VMEM),
        scratch_shapes=scratch,
        compiler_params=pltpu.CompilerParams(collective_id=0),
    )(x, w_mat)
